# baseline (speedup 1.0000x reference)
import functools

import jax
import jax.numpy as jnp
from jax.experimental import pallas as pl
from jax.experimental.pallas import tpu as pltpu

_LN_EPS = 1e-5
_NS_ITERS = 5

_CONTRACT_T = (((1,), (1,)), ((), ()))


def _rcwa_kernel(z_ref, w1_ref, b1_ref, g_ref, bb_ref, w2_ref, b2_ref,
                 wr_ref, br_ref, wo_ref, wv_ref, bv_ref, wm_ref, bm_ref,
                 sc_ref,
                 o_ref,
                 x_scr, s1_scr, s2_scr, m2_scr, c2_scr, w2f_scr, wrf_scr,
                 bo_scr,
                 *, nb1, bt, nrows, d_out):
    j = pl.program_id(0)
    f32 = jnp.float32
    hb = bt // 2

    @pl.when(j == 0)
    def _init():
        s1_scr[...] = jnp.zeros_like(s1_scr)
        s2_scr[...] = jnp.zeros_like(s2_scr)
        wo = wo_ref[...]
        w2f_scr[...] = jax.lax.dot_general(
            wo, w2_ref[...], (((1,), (0,)), ((), ())),
            preferred_element_type=f32)
        wrf_scr[...] = jax.lax.dot_general(
            wo, wr_ref[...], (((1,), (0,)), ((), ())),
            preferred_element_type=f32)
        bo_scr[...] = jax.lax.dot_general(
            b2_ref[...] + br_ref[...], wo, _CONTRACT_T,
            preferred_element_type=f32)

    @pl.when(j < nb1)
    def _stats():
        z = z_ref[...]
        h = jax.lax.dot_general(z, w1_ref[...], _CONTRACT_T,
                                preferred_element_type=f32) + b1_ref[...]
        m = jnp.mean(h, axis=-1, keepdims=True)
        hc = h - m
        v = jnp.mean(hc * hc, axis=-1, keepdims=True)
        h = hc * jax.lax.rsqrt(v + _LN_EPS) * g_ref[...] + bb_ref[...]
        h = 0.5 * h * (1.0 + jax.lax.erf(h * 0.7071067811865476))
        x = (jax.lax.dot_general(h, w2f_scr[...], _CONTRACT_T,
                                 preferred_element_type=f32)
             + jax.lax.dot_general(z, wrf_scr[...], _CONTRACT_T,
                                   preferred_element_type=f32)
             + bo_scr[...])
        xb = x.astype(jnp.bfloat16)
        x_scr[pl.ds(j * hb, hb), 0:d_out] = xb[0:hb]
        x_scr[pl.ds(j * hb, hb), d_out:2 * d_out] = xb[hb:bt]
        s1 = jnp.sum(x, axis=0, keepdims=True)
        s2 = jax.lax.dot_general(x, x, (((0,), (0,)), ((), ())),
                                 preferred_element_type=f32)
        s1_scr[...] += jnp.broadcast_to(s1, s1_scr.shape)
        s2_scr[...] += s2

    @pl.when(j == nb1)
    def _solve():
        s1 = s1_scr[0:1, :]
        s2 = s2_scr[...]
        n = jnp.float32(nrows)
        mu = s1 / n
        outer = jax.lax.dot_general(mu, mu, (((0,), (0,)), ((), ())),
                                    preferred_element_type=f32)
        denom = jnp.float32(nrows - 1 if nrows > 1 else 1)
        ii = jax.lax.broadcasted_iota(jnp.int32, (d_out, d_out), 0)
        jj = jax.lax.broadcasted_iota(jnp.int32, (d_out, d_out), 1)
        eye = jnp.where(ii == jj, 1.0, 0.0).astype(f32)
        sigma = (s2 - n * outer) / denom + 0.001 * eye
        tr = jnp.sum(sigma * eye)
        sn = tr * 1.5 + 1e-6
        ss = sigma / sn
        w = eye
        for _ in range(_NS_ITERS):
            t = jnp.dot(w, ss, preferred_element_type=f32)
            p = jax.lax.dot_general(t, w, _CONTRACT_T,
                                    preferred_element_type=f32)
            w = jnp.dot(1.5 * eye - 0.5 * p, w, preferred_element_type=f32)
        a = w / jnp.sqrt(sn)
        mv = jax.lax.dot_general(a, wv_ref[...], (((0,), (1,)), ((), ())),
                                 preferred_element_type=f32)
        mm = jax.lax.dot_general(a, wm_ref[...], (((0,), (1,)), ((), ())),
                                 preferred_element_type=f32)
        cv = bv_ref[...] - jnp.dot(mu, mv, preferred_element_type=f32)
        cm = bm_ref[...] - jnp.dot(mu, mm, preferred_element_type=f32)
        sc = sc_ref[0, 0]
        m2 = jnp.concatenate([mv * sc, mm], axis=1)
        zer = jnp.zeros_like(m2)
        m2_scr[...] = jnp.concatenate(
            [jnp.concatenate([m2, zer], axis=1),
             jnp.concatenate([zer, m2], axis=1)],
            axis=0).astype(jnp.bfloat16)
        c2 = jnp.concatenate([cv * sc, cm, cv * sc, cm], axis=1)
        c2_scr[...] = jnp.broadcast_to(jnp.transpose(c2), c2_scr.shape)

    @pl.when(j >= nb1)
    def _apply():
        jj2 = j - nb1
        xp = x_scr[pl.ds(jj2 * hb, hb), :]
        yt = jax.lax.dot_general(m2_scr[...], xp, (((0,), (1,)), ((), ())),
                                 preferred_element_type=f32) + c2_scr[:, 0:1]
        o_ref[:, 0:hb] = (yt[0:d_out, :]
                          * jax.nn.sigmoid(yt[d_out:2 * d_out, :]))
        o_ref[:, hb:bt] = (yt[2 * d_out:3 * d_out, :]
                           * jax.nn.sigmoid(yt[3 * d_out:, :]))


def kernel(z_mantis, W1, b1, ln_g, ln_b, W2, b2, Wr, br, Wo, Wv, bv, Wm, bm, scale):
    B, d_in = z_mantis.shape
    d_hid = W1.shape[0]
    d_out = W2.shape[0]
    f32 = jnp.float32

    b1r = b1.reshape(1, d_hid)
    gr = ln_g.reshape(1, d_hid)
    lbr = ln_b.reshape(1, d_hid)
    b2r = b2.reshape(1, d_out)
    brr = br.reshape(1, d_out)
    bvr = bv.reshape(1, d_out)
    bmr = bm.reshape(1, d_out)
    scr = scale.reshape(1, 1)

    bt = 4096 if B % 4096 == 0 else B
    nb1 = B // bt

    full = lambda r, c: pl.BlockSpec((r, c), lambda j: (0, 0))
    out = pl.pallas_call(
        functools.partial(_rcwa_kernel, nb1=nb1, bt=bt, nrows=B, d_out=d_out),
        grid=(2 * nb1,),
        in_specs=[
            pl.BlockSpec((bt, d_in), lambda j: (jnp.minimum(j, nb1 - 1), 0)),
            full(d_hid, d_in),
            full(1, d_hid),
            full(1, d_hid),
            full(1, d_hid),
            full(d_out, d_hid),
            full(1, d_out),
            full(d_out, d_in),
            full(1, d_out),
            full(d_out, d_out),
            full(d_out, d_out),
            full(1, d_out),
            full(d_out, d_out),
            full(1, d_out),
            full(1, 1),
        ],
        out_specs=pl.BlockSpec((d_out, bt),
                               lambda j: (0, jnp.maximum(j - nb1, 0))),
        out_shape=jax.ShapeDtypeStruct((d_out, B), f32),
        scratch_shapes=[
            pltpu.VMEM((B // 2, 2 * d_out), jnp.bfloat16),
            pltpu.VMEM((8, d_out), f32),
            pltpu.VMEM((d_out, d_out), f32),
            pltpu.VMEM((2 * d_out, 4 * d_out), jnp.bfloat16),
            pltpu.VMEM((4 * d_out, 8), f32),
            pltpu.VMEM((d_out, d_hid), f32),
            pltpu.VMEM((d_out, d_in), f32),
            pltpu.VMEM((1, d_out), f32),
        ],
        compiler_params=pltpu.CompilerParams(
            dimension_semantics=("arbitrary",),
            vmem_limit_bytes=56 * 1024 * 1024),
        name="rcwa_fused",
    )(z_mantis, W1, b1r, gr, lbr, W2, b2r, Wr, brr, Wo, Wv, bvr, Wm, bmr, scr)
    return jnp.transpose(out)


# speedup vs baseline: 1.0501x; 1.0501x over previous
import functools

import jax
import jax.numpy as jnp
from jax.experimental import pallas as pl
from jax.experimental.pallas import tpu as pltpu

_LN_EPS = 1e-5
_NS_ITERS = 5

_CONTRACT_T = (((1,), (1,)), ((), ()))


def _rcwa_kernel(z0_ref, z1_ref, w1_ref, b1_ref, g_ref, bb_ref, w2_ref, b2_ref,
                 wr_ref, br_ref, wo_ref, wv_ref, bv_ref, wm_ref, bm_ref,
                 sc_ref,
                 o_ref,
                 x_scr, s1_scr, s2_scr, m2_scr, c2_scr, w2f_scr, wrf_scr,
                 bo_scr,
                 *, nb1, bt, nrows, d_out):
    j = pl.program_id(0)
    f32 = jnp.float32
    hb = bt // 2

    @pl.when(j == 0)
    def _init():
        s1_scr[...] = jnp.zeros_like(s1_scr)
        s2_scr[...] = jnp.zeros_like(s2_scr)
        wo = wo_ref[...]
        w2f_scr[...] = jax.lax.dot_general(
            wo, w2_ref[...], (((1,), (0,)), ((), ())),
            preferred_element_type=f32)
        wrf_scr[...] = jax.lax.dot_general(
            wo, wr_ref[...], (((1,), (0,)), ((), ())),
            preferred_element_type=f32)
        bo_scr[...] = jax.lax.dot_general(
            b2_ref[...] + br_ref[...], wo, _CONTRACT_T,
            preferred_element_type=f32)

    @pl.when(j < nb1)
    def _stats():
        def _half(z):
            h = jax.lax.dot_general(z, w1_ref[...], _CONTRACT_T,
                                    preferred_element_type=f32) + b1_ref[...]
            m = jnp.mean(h, axis=-1, keepdims=True)
            hc = h - m
            v = jnp.mean(hc * hc, axis=-1, keepdims=True)
            h = hc * jax.lax.rsqrt(v + _LN_EPS) * g_ref[...] + bb_ref[...]
            h = 0.5 * h * (1.0 + jax.lax.erf(h * 0.7071067811865476))
            return (jax.lax.dot_general(h, w2f_scr[...], _CONTRACT_T,
                                        preferred_element_type=f32)
                    + jax.lax.dot_general(z, wrf_scr[...], _CONTRACT_T,
                                          preferred_element_type=f32)
                    + bo_scr[...])

        x0 = _half(z0_ref[...])
        x_scr[pl.ds(j * hb, hb), 0:d_out] = x0.astype(jnp.bfloat16)
        x1 = _half(z1_ref[...])
        x_scr[pl.ds(j * hb, hb), d_out:2 * d_out] = x1.astype(jnp.bfloat16)
        s1 = (jnp.sum(x0, axis=0, keepdims=True)
              + jnp.sum(x1, axis=0, keepdims=True))
        s2 = (jax.lax.dot_general(x0, x0, (((0,), (0,)), ((), ())),
                                  preferred_element_type=f32)
              + jax.lax.dot_general(x1, x1, (((0,), (0,)), ((), ())),
                                    preferred_element_type=f32))
        s1_scr[...] += jnp.broadcast_to(s1, s1_scr.shape)
        s2_scr[...] += s2

    @pl.when(j == nb1)
    def _solve():
        s1 = s1_scr[0:1, :]
        s2 = s2_scr[...]
        n = jnp.float32(nrows)
        mu = s1 / n
        outer = jax.lax.dot_general(mu, mu, (((0,), (0,)), ((), ())),
                                    preferred_element_type=f32)
        denom = jnp.float32(nrows - 1 if nrows > 1 else 1)
        ii = jax.lax.broadcasted_iota(jnp.int32, (d_out, d_out), 0)
        jj = jax.lax.broadcasted_iota(jnp.int32, (d_out, d_out), 1)
        eye = jnp.where(ii == jj, 1.0, 0.0).astype(f32)
        sigma = (s2 - n * outer) / denom + 0.001 * eye
        tr = jnp.sum(sigma * eye)
        sn = tr * 1.5 + 1e-6
        ss = sigma / sn
        w = eye
        for _ in range(_NS_ITERS):
            t = jnp.dot(w, ss, preferred_element_type=f32)
            p = jax.lax.dot_general(t, w, _CONTRACT_T,
                                    preferred_element_type=f32)
            w = jnp.dot(1.5 * eye - 0.5 * p, w, preferred_element_type=f32)
        a = w / jnp.sqrt(sn)
        mv = jax.lax.dot_general(a, wv_ref[...], (((0,), (1,)), ((), ())),
                                 preferred_element_type=f32)
        mm = jax.lax.dot_general(a, wm_ref[...], (((0,), (1,)), ((), ())),
                                 preferred_element_type=f32)
        cv = bv_ref[...] - jnp.dot(mu, mv, preferred_element_type=f32)
        cm = bm_ref[...] - jnp.dot(mu, mm, preferred_element_type=f32)
        sc = sc_ref[0, 0]
        m2 = jnp.concatenate([mv * sc, mm], axis=1)
        zer = jnp.zeros_like(m2)
        m2_scr[...] = jnp.concatenate(
            [jnp.concatenate([m2, zer], axis=1),
             jnp.concatenate([zer, m2], axis=1)],
            axis=0).astype(jnp.bfloat16)
        c2 = jnp.concatenate([cv * sc, cm, cv * sc, cm], axis=1)
        c2_scr[...] = jnp.broadcast_to(jnp.transpose(c2), c2_scr.shape)

    @pl.when(j >= nb1)
    def _apply():
        jj2 = j - nb1
        xp = x_scr[pl.ds(jj2 * hb, hb), :]
        yt = jax.lax.dot_general(m2_scr[...], xp, (((0,), (1,)), ((), ())),
                                 preferred_element_type=f32) + c2_scr[:, 0:1]
        o_ref[:, 0:hb] = (yt[0:d_out, :]
                          * jax.nn.sigmoid(yt[d_out:2 * d_out, :]))
        o_ref[:, hb:bt] = (yt[2 * d_out:3 * d_out, :]
                           * jax.nn.sigmoid(yt[3 * d_out:, :]))


def kernel(z_mantis, W1, b1, ln_g, ln_b, W2, b2, Wr, br, Wo, Wv, bv, Wm, bm, scale):
    B, d_in = z_mantis.shape
    d_hid = W1.shape[0]
    d_out = W2.shape[0]
    f32 = jnp.float32

    b1r = b1.reshape(1, d_hid)
    gr = ln_g.reshape(1, d_hid)
    lbr = ln_b.reshape(1, d_hid)
    b2r = b2.reshape(1, d_out)
    brr = br.reshape(1, d_out)
    bvr = bv.reshape(1, d_out)
    bmr = bm.reshape(1, d_out)
    scr = scale.reshape(1, 1)

    bt = 4096 if B % 4096 == 0 else B
    nb1 = B // bt

    full = lambda r, c: pl.BlockSpec((r, c), lambda j: (0, 0))
    out = pl.pallas_call(
        functools.partial(_rcwa_kernel, nb1=nb1, bt=bt, nrows=B, d_out=d_out),
        grid=(2 * nb1,),
        in_specs=[
            pl.BlockSpec((bt // 2, d_in),
                         lambda j: (2 * jnp.minimum(j, nb1 - 1), 0)),
            pl.BlockSpec((bt // 2, d_in),
                         lambda j: (2 * jnp.minimum(j, nb1 - 1) + 1, 0)),
            full(d_hid, d_in),
            full(1, d_hid),
            full(1, d_hid),
            full(1, d_hid),
            full(d_out, d_hid),
            full(1, d_out),
            full(d_out, d_in),
            full(1, d_out),
            full(d_out, d_out),
            full(d_out, d_out),
            full(1, d_out),
            full(d_out, d_out),
            full(1, d_out),
            full(1, 1),
        ],
        out_specs=pl.BlockSpec((d_out, bt),
                               lambda j: (0, jnp.maximum(j - nb1, 0))),
        out_shape=jax.ShapeDtypeStruct((d_out, B), f32),
        scratch_shapes=[
            pltpu.VMEM((B // 2, 2 * d_out), jnp.bfloat16),
            pltpu.VMEM((8, d_out), f32),
            pltpu.VMEM((d_out, d_out), f32),
            pltpu.VMEM((2 * d_out, 4 * d_out), jnp.bfloat16),
            pltpu.VMEM((4 * d_out, 8), f32),
            pltpu.VMEM((d_out, d_hid), f32),
            pltpu.VMEM((d_out, d_in), f32),
            pltpu.VMEM((1, d_out), f32),
        ],
        compiler_params=pltpu.CompilerParams(
            dimension_semantics=("arbitrary",),
            vmem_limit_bytes=56 * 1024 * 1024),
        name="rcwa_fused",
    )(z_mantis, z_mantis, W1, b1r, gr, lbr, W2, b2r, Wr, brr, Wo, Wv, bvr,
      Wm, bmr, scr)
    return jnp.transpose(out)
